# baseline (speedup 1.0000x reference)
import functools

import jax
import jax.numpy as jnp
from jax import lax
from jax.experimental import pallas as pl
from jax.experimental.pallas import tpu as pltpu
from jax.experimental.pallas import tpu_sc as plsc

_CH = 128


def _sc_gather(table, idx3, B, D, nw, nch):
    bpw = B // nw

    @functools.partial(
        pl.kernel,
        mesh=plsc.VectorSubcoreMesh(core_axis_name="c", subcore_axis_name="s"),
        out_type=jax.ShapeDtypeStruct((B, D), jnp.float32),
        scratch_types=[
            pltpu.VMEM((nch, _CH), jnp.int32),
            pltpu.VMEM((bpw, D), jnp.float32),
            pltpu.SemaphoreType.DMA,
        ],
        compiler_params=pltpu.CompilerParams(use_tc_tiling_on_sc=False),
    )
    def gather_k(table_hbm, idx_hbm, out_hbm, idx_v, rows_v, sem):
        nc = 2
        wid = lax.axis_index("s") * nc + lax.axis_index("c")
        pltpu.sync_copy(idx_hbm.at[wid], idx_v)
        copies = [
            pltpu.async_copy(
                table_hbm.at[idx_v.at[j]],
                rows_v.at[pl.ds(j * _CH, _CH)],
                sem,
            )
            for j in range(nch)
        ]
        for c in copies:
            c.wait()
        pltpu.sync_copy(rows_v, out_hbm.at[pl.ds(wid * bpw, bpw)])

    return gather_k(table, idx3)


def _mlp_body(x_ref, e_ref, w0x_ref, w0e_ref, b0_ref, w1_ref, b1_ref,
              w2_ref, b2_ref, wt_ref, bt_ref, o_ref):
    f32 = jnp.float32
    h = jnp.dot(x_ref[...], w0x_ref[...], preferred_element_type=f32,
                precision=lax.Precision.HIGHEST)
    h += jnp.dot(e_ref[...], w0e_ref[...], preferred_element_type=f32,
                 precision=lax.Precision.HIGHEST)
    h = jnp.maximum(h + b0_ref[...], 0.0)
    h = h + jnp.maximum(
        jnp.dot(h, w1_ref[...], preferred_element_type=f32,
                precision=lax.Precision.HIGHEST) + b1_ref[...], 0.0)
    h = h + jnp.maximum(
        jnp.dot(h, w2_ref[...], preferred_element_type=f32,
                precision=lax.Precision.HIGHEST) + b2_ref[...], 0.0)
    o_ref[...] = jnp.dot(h, wt_ref[...], preferred_element_type=f32,
                         precision=lax.Precision.HIGHEST) + bt_ref[...]


def kernel(x, table, W0, b0, W1, b1, W2, b2, Wt, bt):
    B, C = x.shape
    D = table.shape[1]
    H = W0.shape[1]
    HP = 128

    info = plsc.get_sparse_core_info()
    nw = info.num_cores * info.num_subcores
    bpw = B // nw
    nch = bpw // _CH

    idx = x[:, 0].astype(jnp.int32)
    idx3 = idx.reshape(nw, nch, _CH)
    emb = _sc_gather(table, idx3, B, D, nw, nch)

    ph = HP - H
    w0x = jnp.pad(W0[D:], ((1, 0), (0, ph)))
    w0e = jnp.pad(W0[:D], ((0, 0), (0, ph)))
    w1 = jnp.pad(W1, ((0, ph), (0, ph)))
    w2 = jnp.pad(W2, ((0, ph), (0, ph)))
    wt = jnp.pad(Wt, ((0, ph), (0, 0)))
    b0p = jnp.pad(b0, (0, ph)).reshape(1, HP)
    b1p = jnp.pad(b1, (0, ph)).reshape(1, HP)
    b2p = jnp.pad(b2, (0, ph)).reshape(1, HP)
    btp = bt.reshape(1, 1)

    BB = 2048
    nb = B // BB
    rep = lambda i: (0, 0)
    out = pl.pallas_call(
        _mlp_body,
        grid=(nb,),
        in_specs=[
            pl.BlockSpec((BB, C), lambda i: (i, 0)),
            pl.BlockSpec((BB, D), lambda i: (i, 0)),
            pl.BlockSpec((C, HP), rep),
            pl.BlockSpec((D, HP), rep),
            pl.BlockSpec((1, HP), rep),
            pl.BlockSpec((HP, HP), rep),
            pl.BlockSpec((1, HP), rep),
            pl.BlockSpec((HP, HP), rep),
            pl.BlockSpec((1, HP), rep),
            pl.BlockSpec((HP, 1), rep),
            pl.BlockSpec((1, 1), rep),
        ],
        out_specs=pl.BlockSpec((BB, 1), lambda i: (i, 0)),
        out_shape=jax.ShapeDtypeStruct((B, 1), jnp.float32),
    )(x, emb, w0x, w0e, b0p, w1, b1p, w2, b2p, wt, btp)
    return out


# speedup vs baseline: 1.5798x; 1.5798x over previous
import functools

import jax
import jax.numpy as jnp
from jax import lax
from jax.experimental import pallas as pl
from jax.experimental.pallas import tpu as pltpu
from jax.experimental.pallas import tpu_sc as plsc


def _sc_gather(table, idx, B, D, nw):
    bpw = B // nw

    @functools.partial(
        pl.kernel,
        mesh=plsc.VectorSubcoreMesh(core_axis_name="c", subcore_axis_name="s"),
        out_type=jax.ShapeDtypeStruct((B, D), jnp.float32),
        scratch_types=[
            pltpu.VMEM((bpw,), jnp.int32),
            pltpu.VMEM((bpw, D), jnp.float32),
            pltpu.SemaphoreType.DMA,
        ],
        compiler_params=pltpu.CompilerParams(needs_layout_passes=False),
    )
    def gather_k(table_hbm, idx_hbm, out_hbm, idx_v, rows_v, sem):
        nc = 2
        wid = lax.axis_index("s") * nc + lax.axis_index("c")
        base = wid * bpw
        pltpu.sync_copy(idx_hbm.at[pl.ds(base, bpw)], idx_v)

        lanes = lax.iota(jnp.int32, 16)
        zeros = jnp.zeros((16,), jnp.int32)

        def issue(g, _):
            v = idx_v[pl.ds(g * 16, 16)]
            for l in range(16):
                i = lax.reduce_sum(jnp.where(lanes == l, v, zeros), axes=(0,))
                pltpu.async_copy(table_hbm.at[i], rows_v.at[g * 16 + l], sem)
            return 0

        lax.fori_loop(0, bpw // 16, issue, 0)

        def drain(j, _):
            pltpu.make_async_copy(table_hbm.at[0], rows_v.at[0], sem).wait()
            return 0

        lax.fori_loop(0, bpw, drain, 0)
        pltpu.sync_copy(rows_v, out_hbm.at[pl.ds(base, bpw)])

    return gather_k(table, idx)


def _mlp_body(x_ref, e_ref, w0x_ref, w0e_ref, b0_ref, w1_ref, b1_ref,
              w2_ref, b2_ref, wt_ref, bt_ref, o_ref):
    f32 = jnp.float32
    h = jnp.dot(x_ref[...], w0x_ref[...], preferred_element_type=f32,
                precision=lax.Precision.HIGHEST)
    h += jnp.dot(e_ref[...], w0e_ref[...], preferred_element_type=f32,
                 precision=lax.Precision.HIGHEST)
    h = jnp.maximum(h + b0_ref[...], 0.0)
    h = h + jnp.maximum(
        jnp.dot(h, w1_ref[...], preferred_element_type=f32,
                precision=lax.Precision.HIGHEST) + b1_ref[...], 0.0)
    h = h + jnp.maximum(
        jnp.dot(h, w2_ref[...], preferred_element_type=f32,
                precision=lax.Precision.HIGHEST) + b2_ref[...], 0.0)
    o_ref[...] = jnp.dot(h, wt_ref[...], preferred_element_type=f32,
                         precision=lax.Precision.HIGHEST) + bt_ref[...]


def kernel(x, table, W0, b0, W1, b1, W2, b2, Wt, bt):
    B, C = x.shape
    D = table.shape[1]
    H = W0.shape[1]
    HP = 128

    info = plsc.get_sparse_core_info()
    nw = info.num_cores * info.num_subcores

    idx = x[:, 0].astype(jnp.int32)
    emb = _sc_gather(table, idx, B, D, nw)

    ph = HP - H
    w0x = jnp.pad(W0[D:], ((1, 0), (0, ph)))
    w0e = jnp.pad(W0[:D], ((0, 0), (0, ph)))
    w1 = jnp.pad(W1, ((0, ph), (0, ph)))
    w2 = jnp.pad(W2, ((0, ph), (0, ph)))
    wt = jnp.pad(Wt, ((0, ph), (0, 0)))
    b0p = jnp.pad(b0, (0, ph)).reshape(1, HP)
    b1p = jnp.pad(b1, (0, ph)).reshape(1, HP)
    b2p = jnp.pad(b2, (0, ph)).reshape(1, HP)
    btp = bt.reshape(1, 1)

    BB = 2048
    nb = B // BB
    rep = lambda i: (0, 0)
    out = pl.pallas_call(
        _mlp_body,
        grid=(nb,),
        in_specs=[
            pl.BlockSpec((BB, C), lambda i: (i, 0)),
            pl.BlockSpec((BB, D), lambda i: (i, 0)),
            pl.BlockSpec((C, HP), rep),
            pl.BlockSpec((D, HP), rep),
            pl.BlockSpec((1, HP), rep),
            pl.BlockSpec((HP, HP), rep),
            pl.BlockSpec((1, HP), rep),
            pl.BlockSpec((HP, HP), rep),
            pl.BlockSpec((1, HP), rep),
            pl.BlockSpec((HP, 1), rep),
            pl.BlockSpec((1, 1), rep),
        ],
        out_specs=pl.BlockSpec((BB, 1), lambda i: (i, 0)),
        out_shape=jax.ShapeDtypeStruct((B, 1), jnp.float32),
    )(x, emb, w0x, w0e, b0p, w1, b1p, w2, b2p, wt, btp)
    return out


# speedup vs baseline: 1.5873x; 1.0048x over previous
import functools

import jax
import jax.numpy as jnp
from jax import lax
from jax.experimental import pallas as pl
from jax.experimental.pallas import tpu as pltpu
from jax.experimental.pallas import tpu_sc as plsc


def _sc_gather(table, idx, B, D, nw):
    bpw = B // nw

    @functools.partial(
        pl.kernel,
        mesh=plsc.VectorSubcoreMesh(core_axis_name="c", subcore_axis_name="s"),
        out_type=jax.ShapeDtypeStruct((B, D), jnp.float32),
        scratch_types=[
            pltpu.VMEM((bpw,), jnp.int32),
            pltpu.VMEM((bpw, D), jnp.float32),
            pltpu.SemaphoreType.DMA,
        ],
    )
    def gather_k(table_hbm, idx_hbm, out_hbm, idx_v, rows_v, sem):
        nc = 2
        wid = lax.axis_index("s") * nc + lax.axis_index("c")
        base = wid * bpw
        pltpu.sync_copy(idx_hbm.at[pl.ds(base, bpw)], idx_v)

        def issue(g, _):
            v = idx_v[pl.ds(g * 16, 16)]
            for l in range(16):
                i = v[l]
                pltpu.async_copy(table_hbm.at[i], rows_v.at[g * 16 + l], sem)
            return 0

        lax.fori_loop(0, bpw // 16, issue, 0)

        def drain(j, _):
            pltpu.make_async_copy(table_hbm.at[0], rows_v.at[0], sem).wait()
            return 0

        lax.fori_loop(0, bpw, drain, 0)
        pltpu.sync_copy(rows_v, out_hbm.at[pl.ds(base, bpw)])

    return gather_k(table, idx)


def _mlp_body(x_ref, e_ref, w0x_ref, w0e_ref, b0_ref, w1_ref, b1_ref,
              w2_ref, b2_ref, wt_ref, bt_ref, o_ref):
    f32 = jnp.float32
    h = jnp.dot(x_ref[...], w0x_ref[...], preferred_element_type=f32,
                precision=lax.Precision.HIGHEST)
    h += jnp.dot(e_ref[...], w0e_ref[...], preferred_element_type=f32,
                 precision=lax.Precision.HIGHEST)
    h = jnp.maximum(h + b0_ref[...], 0.0)
    h = h + jnp.maximum(
        jnp.dot(h, w1_ref[...], preferred_element_type=f32,
                precision=lax.Precision.HIGHEST) + b1_ref[...], 0.0)
    h = h + jnp.maximum(
        jnp.dot(h, w2_ref[...], preferred_element_type=f32,
                precision=lax.Precision.HIGHEST) + b2_ref[...], 0.0)
    o_ref[...] = jnp.dot(h, wt_ref[...], preferred_element_type=f32,
                         precision=lax.Precision.HIGHEST) + bt_ref[...]


def kernel(x, table, W0, b0, W1, b1, W2, b2, Wt, bt):
    B, C = x.shape
    D = table.shape[1]
    H = W0.shape[1]
    HP = 128

    info = plsc.get_sparse_core_info()
    nw = info.num_cores * info.num_subcores

    idx = x[:, 0].astype(jnp.int32)
    emb = _sc_gather(table, idx, B, D, nw)

    ph = HP - H
    w0x = jnp.pad(W0[D:], ((1, 0), (0, ph)))
    w0e = jnp.pad(W0[:D], ((0, 0), (0, ph)))
    w1 = jnp.pad(W1, ((0, ph), (0, ph)))
    w2 = jnp.pad(W2, ((0, ph), (0, ph)))
    wt = jnp.pad(Wt, ((0, ph), (0, 0)))
    b0p = jnp.pad(b0, (0, ph)).reshape(1, HP)
    b1p = jnp.pad(b1, (0, ph)).reshape(1, HP)
    b2p = jnp.pad(b2, (0, ph)).reshape(1, HP)
    btp = bt.reshape(1, 1)

    BB = 2048
    nb = B // BB
    rep = lambda i: (0, 0)
    out = pl.pallas_call(
        _mlp_body,
        grid=(nb,),
        in_specs=[
            pl.BlockSpec((BB, C), lambda i: (i, 0)),
            pl.BlockSpec((BB, D), lambda i: (i, 0)),
            pl.BlockSpec((C, HP), rep),
            pl.BlockSpec((D, HP), rep),
            pl.BlockSpec((1, HP), rep),
            pl.BlockSpec((HP, HP), rep),
            pl.BlockSpec((1, HP), rep),
            pl.BlockSpec((HP, HP), rep),
            pl.BlockSpec((1, HP), rep),
            pl.BlockSpec((HP, 1), rep),
            pl.BlockSpec((1, 1), rep),
        ],
        out_specs=pl.BlockSpec((BB, 1), lambda i: (i, 0)),
        out_shape=jax.ShapeDtypeStruct((B, 1), jnp.float32),
    )(x, emb, w0x, w0e, b0p, w1, b1p, w2, b2p, wt, btp)
    return out
